# baseline (speedup 1.0000x reference)
import functools

import jax
import jax.numpy as jnp
from jax import lax
from jax.experimental import pallas as pl
from jax.experimental.pallas import tpu as pltpu
from jax.experimental.pallas import tpu_sc as plsc

F = 26
V = 100000
D = 16
B = 16384
NC = 2
NS = 16
NW = NC * NS
CH = 128
BLKS = B // CH
BLKS_PER_W = BLKS // NW


def _sc_body(*refs):
    feat_refs = refs[:F]
    w_refs = refs[F:2 * F]
    out_ref = refs[2 * F]
    idx_v, rows_v, sem = refs[2 * F + 1:]

    wid = lax.axis_index("s") * NC + lax.axis_index("c")
    blk0 = wid * BLKS_PER_W

    for f in range(F):
        pltpu.sync_copy(feat_refs[f].at[pl.ds(blk0, BLKS_PER_W)], idx_v)

        def inner(j, carry, f=f):
            pltpu.async_copy(w_refs[f].at[idx_v.at[j]], rows_v, sem).wait()
            pltpu.sync_copy(
                rows_v,
                out_ref.at[pl.ds((blk0 + j) * CH, CH), pl.ds(f * D, D)],
            )
            return carry

        lax.fori_loop(0, BLKS_PER_W, inner, 0)


@jax.jit
def _run(feats, tables):
    kern = functools.partial(
        pl.kernel,
        out_type=jax.ShapeDtypeStruct((B, F * D), jnp.float32),
        mesh=plsc.VectorSubcoreMesh(core_axis_name="c", subcore_axis_name="s"),
        scratch_types=[
            pltpu.VMEM((BLKS_PER_W, CH), jnp.int32),
            pltpu.VMEM((CH, D), jnp.float32),
            pltpu.SemaphoreType.DMA,
        ],
        compiler_params=pltpu.CompilerParams(use_tc_tiling_on_sc=False),
    )(_sc_body)
    return kern(*feats, *tables)


def kernel(feat_0, feat_1, feat_2, feat_3, feat_4, feat_5, feat_6, feat_7,
           feat_8, feat_9, feat_10, feat_11, feat_12, feat_13, feat_14,
           feat_15, feat_16, feat_17, feat_18, feat_19, feat_20, feat_21,
           feat_22, feat_23, feat_24, feat_25,
           W_0, W_1, W_2, W_3, W_4, W_5, W_6, W_7, W_8, W_9, W_10, W_11,
           W_12, W_13, W_14, W_15, W_16, W_17, W_18, W_19, W_20, W_21,
           W_22, W_23, W_24, W_25):
    feats = [feat_0, feat_1, feat_2, feat_3, feat_4, feat_5, feat_6, feat_7,
             feat_8, feat_9, feat_10, feat_11, feat_12, feat_13, feat_14,
             feat_15, feat_16, feat_17, feat_18, feat_19, feat_20, feat_21,
             feat_22, feat_23, feat_24, feat_25]
    tables = [W_0, W_1, W_2, W_3, W_4, W_5, W_6, W_7, W_8, W_9, W_10, W_11,
              W_12, W_13, W_14, W_15, W_16, W_17, W_18, W_19, W_20, W_21,
              W_22, W_23, W_24, W_25]
    feats = [f.reshape(BLKS, CH) for f in feats]
    return _run(feats, tables)


# speedup vs baseline: 1.0305x; 1.0305x over previous
import functools

import jax
import jax.numpy as jnp
from jax import lax
from jax.experimental import pallas as pl
from jax.experimental.pallas import tpu as pltpu
from jax.experimental.pallas import tpu_sc as plsc

F = 26
V = 100000
D = 16
B = 16384
NC = 2
NS = 16
NW = NC * NS
CH = 128
ROWS_PER_W = B // NW
BLKS_PER_W = ROWS_PER_W // CH


NBUF = 4
LAG = 2


def _sc_body(*refs):
    feat_refs = refs[:F]
    w_refs = refs[F:2 * F]
    out_ref = refs[2 * F]
    idx_v = refs[2 * F + 1]
    bufs = refs[2 * F + 2:2 * F + 2 + NBUF]
    gsem = refs[2 * F + 2 + NBUF]
    ssem = refs[2 * F + 3 + NBUF]

    wid = lax.axis_index("s") * NC + lax.axis_index("c")
    base = pl.multiple_of(wid * ROWS_PER_W, ROWS_PER_W)

    icopies = [
        pltpu.async_copy(feat_refs[f].at[pl.ds(base, ROWS_PER_W)],
                         idx_v.at[f], gsem)
        for f in range(F)
    ]
    for c in icopies:
        c.wait()

    steps = [(f, j) for f in range(F) for j in range(BLKS_PER_W)]
    total = len(steps)
    gathers, stores = {}, {}

    def issue_gather(t):
        f, j = steps[t]
        gathers[t] = pltpu.async_copy(
            w_refs[f].at[idx_v.at[f, pl.ds(j * CH, CH)]],
            bufs[t % NBUF],
            gsem,
        )

    def issue_store(t):
        f, j = steps[t]
        gathers[t].wait()
        stores[t] = pltpu.async_copy(
            bufs[t % NBUF],
            out_ref.at[pl.ds(base + j * CH, CH), pl.ds(f * D, D)],
            ssem,
        )

    for t in range(LAG):
        issue_gather(t)
    for t in range(total):
        issue_store(t)
        nxt = t + LAG
        if nxt < total:
            if nxt >= NBUF:
                stores[nxt - NBUF].wait()
            issue_gather(nxt)
    for t in range(total - NBUF, total):
        stores[t].wait()


@jax.jit
def _run(feats, tables):
    kern = functools.partial(
        pl.kernel,
        out_type=jax.ShapeDtypeStruct((B, F * D), jnp.float32),
        mesh=plsc.VectorSubcoreMesh(core_axis_name="c", subcore_axis_name="s"),
        scratch_types=(
            [pltpu.VMEM((F, ROWS_PER_W), jnp.int32)]
            + [pltpu.VMEM((CH, D), jnp.float32) for _ in range(NBUF)]
            + [pltpu.SemaphoreType.DMA, pltpu.SemaphoreType.DMA]
        ),
        compiler_params=pltpu.CompilerParams(use_tc_tiling_on_sc=False),
    )(_sc_body)
    return kern(*feats, *tables)


def kernel(feat_0, feat_1, feat_2, feat_3, feat_4, feat_5, feat_6, feat_7,
           feat_8, feat_9, feat_10, feat_11, feat_12, feat_13, feat_14,
           feat_15, feat_16, feat_17, feat_18, feat_19, feat_20, feat_21,
           feat_22, feat_23, feat_24, feat_25,
           W_0, W_1, W_2, W_3, W_4, W_5, W_6, W_7, W_8, W_9, W_10, W_11,
           W_12, W_13, W_14, W_15, W_16, W_17, W_18, W_19, W_20, W_21,
           W_22, W_23, W_24, W_25):
    feats = [feat_0, feat_1, feat_2, feat_3, feat_4, feat_5, feat_6, feat_7,
             feat_8, feat_9, feat_10, feat_11, feat_12, feat_13, feat_14,
             feat_15, feat_16, feat_17, feat_18, feat_19, feat_20, feat_21,
             feat_22, feat_23, feat_24, feat_25]
    tables = [W_0, W_1, W_2, W_3, W_4, W_5, W_6, W_7, W_8, W_9, W_10, W_11,
              W_12, W_13, W_14, W_15, W_16, W_17, W_18, W_19, W_20, W_21,
              W_22, W_23, W_24, W_25]
    return _run(feats, tables)


# speedup vs baseline: 1.3333x; 1.2938x over previous
import functools

import jax
import jax.numpy as jnp
from jax import lax
from jax.experimental import pallas as pl
from jax.experimental.pallas import tpu as pltpu
from jax.experimental.pallas import tpu_sc as plsc

F = 26
V = 100000
D = 16
B = 16384
NC = 2
NS = 16
FPC = F // NC
CW = 6248
SLABW = CW + 32
CAP = 544
FLUSH = 512
NITER = B // 16


def _sc_body(*refs):
    feat_refs = refs[:F]
    wt_refs = refs[F:2 * F]
    out_ref = refs[2 * F]
    slab = refs[2 * F + 1]
    idx_v = refs[2 * F + 2]
    hit_i = refs[2 * F + 3]
    hit_b = refs[2 * F + 4]
    stage = refs[2 * F + 5]
    sidx = refs[2 * F + 6]

    c = lax.axis_index("c")
    s = lax.axis_index("s")
    lo = pl.multiple_of(s * CW, 8)
    hi = lo + CW + jnp.where(s == NS - 1, 32, 0)

    iota16 = lax.iota(jnp.int32, 16)

    def do_flush(f, nrows):
        def gather_16(k, carry):
            off = pl.multiple_of(k * 16, 16)
            rel16 = hit_i[pl.ds(off, 16)]
            pos16 = off + iota16
            for d in range(D):
                vals = plsc.load_gather(
                    slab, [jnp.full((16,), d, jnp.int32), rel16])
                plsc.store_scatter(
                    stage, [pos16, jnp.full((16,), d, jnp.int32)], vals)
            return carry

        lax.fori_loop(0, nrows // 16, gather_16, 0)

        def sidx_one(k, carry):
            pos = k * 16 + iota16
            rows = hit_b[pl.ds(pl.multiple_of(k * 16, 16), 16)] * F + f
            plsc.store_scatter(sidx, [pos >> 7, pos & 127], rows)
            return carry

        lax.fori_loop(0, FLUSH // 16, sidx_one, 0)

        for k in range(FLUSH // 128):
            @pl.when(k * 128 < nrows)
            def _():
                pltpu.sync_copy(stage.at[pl.ds(k * 128, 128)],
                                out_ref.at[sidx.at[k]])

    for f in range(F):
        @pl.when(c == f // FPC)
        def _(f=f):
            wt = wt_refs[f]
            pltpu.sync_copy(wt.at[:, pl.ds(lo, SLABW)], slab)
            pltpu.sync_copy(feat_refs[f], idx_v)

            def scan_iter(i, nh):
                off = pl.multiple_of(i * 16, 16)
                v = idx_v[pl.ds(off, 16)]
                m = (v >= lo) & (v < hi)
                mi = m.astype(jnp.int32)
                pos = nh + jnp.cumsum(mi) - mi
                plsc.store_scatter(hit_i, [pos], v - lo, mask=m)
                plsc.store_scatter(hit_b, [pos], i * 16 + iota16, mask=m)
                nh = nh + jnp.sum(mi)

                @pl.when(nh >= FLUSH)
                def _():
                    do_flush(f, FLUSH)
                    for t in range(2):
                        src = pl.ds(FLUSH + t * 16, 16)
                        dst = t * 16 + iota16
                        plsc.store_scatter(hit_i, [dst], hit_i[src])
                        plsc.store_scatter(hit_b, [dst], hit_b[src])

                return jnp.where(nh >= FLUSH, nh - FLUSH, nh)

            nh = lax.fori_loop(0, NITER, scan_iter, 0)

            @pl.when(nh > 0)
            def _():
                lane0 = iota16 == 0
                h0i = jnp.full(
                    (16,), jnp.sum(jnp.where(lane0, hit_i[pl.ds(0, 16)], 0)),
                    jnp.int32)
                h0b = jnp.full(
                    (16,), jnp.sum(jnp.where(lane0, hit_b[pl.ds(0, 16)], 0)),
                    jnp.int32)

                def pad_one(k, carry):
                    pos = k * 16 + iota16
                    pm = pos >= nh
                    plsc.store_scatter(hit_i, [pos], h0i, mask=pm)
                    plsc.store_scatter(hit_b, [pos], h0b, mask=pm)
                    return carry

                lax.fori_loop(0, FLUSH // 16, pad_one, 0)
                do_flush(f, ((nh + 127) >> 7) << 7)


@jax.jit
def _run(feats, wts):
    kern = functools.partial(
        pl.kernel,
        out_type=jax.ShapeDtypeStruct((F * B, D), jnp.float32),
        mesh=plsc.VectorSubcoreMesh(core_axis_name="c", subcore_axis_name="s"),
        scratch_types=[
            pltpu.VMEM((D, SLABW), jnp.float32),
            pltpu.VMEM((B,), jnp.int32),
            pltpu.VMEM((CAP,), jnp.int32),
            pltpu.VMEM((CAP,), jnp.int32),
            pltpu.VMEM((FLUSH, D), jnp.float32),
            pltpu.VMEM((FLUSH // 128, 128), jnp.int32),
        ],
        compiler_params=pltpu.CompilerParams(use_tc_tiling_on_sc=False, needs_layout_passes=False),
    )(_sc_body)
    out2 = kern(*feats, *wts)
    return out2.reshape(B, F * D)


def kernel(feat_0, feat_1, feat_2, feat_3, feat_4, feat_5, feat_6, feat_7,
           feat_8, feat_9, feat_10, feat_11, feat_12, feat_13, feat_14,
           feat_15, feat_16, feat_17, feat_18, feat_19, feat_20, feat_21,
           feat_22, feat_23, feat_24, feat_25,
           W_0, W_1, W_2, W_3, W_4, W_5, W_6, W_7, W_8, W_9, W_10, W_11,
           W_12, W_13, W_14, W_15, W_16, W_17, W_18, W_19, W_20, W_21,
           W_22, W_23, W_24, W_25):
    feats = [feat_0, feat_1, feat_2, feat_3, feat_4, feat_5, feat_6, feat_7,
             feat_8, feat_9, feat_10, feat_11, feat_12, feat_13, feat_14,
             feat_15, feat_16, feat_17, feat_18, feat_19, feat_20, feat_21,
             feat_22, feat_23, feat_24, feat_25]
    tables = [W_0, W_1, W_2, W_3, W_4, W_5, W_6, W_7, W_8, W_9, W_10, W_11,
              W_12, W_13, W_14, W_15, W_16, W_17, W_18, W_19, W_20, W_21,
              W_22, W_23, W_24, W_25]
    wts = [jnp.swapaxes(w, 0, 1) for w in tables]
    return _run(feats, wts)


# speedup vs baseline: 1.3836x; 1.0378x over previous
import functools

import jax
import jax.numpy as jnp
from jax import lax
from jax.experimental import pallas as pl
from jax.experimental.pallas import tpu as pltpu
from jax.experimental.pallas import tpu_sc as plsc

F = 26
V = 100000
D = 16
B = 16384
NC = 2
NS = 16
FPC = F // NC
CW = 6248
SLABW = CW + 32
CAP = 544
FLUSH = 512
NITER = B // 16


def _sc_body(*refs):
    feat_refs = refs[:F]
    wt_refs = refs[F:2 * F]
    out_ref = refs[2 * F]
    slab = refs[2 * F + 1]
    idx_v = refs[2 * F + 2]
    hit_i = refs[2 * F + 3]
    hit_b = refs[2 * F + 4]
    stage = refs[2 * F + 5]
    sidx = refs[2 * F + 6]

    c = lax.axis_index("c")
    s = lax.axis_index("s")
    lo = pl.multiple_of(s * CW, 8)
    hi = lo + CW + jnp.where(s == NS - 1, 32, 0)

    iota16 = lax.iota(jnp.int32, 16)

    def do_flush(f, nrows):
        def gather_16(k, carry):
            off = pl.multiple_of(k * 16, 16)
            rel16 = hit_i[pl.ds(off, 16)]
            pos16 = off + iota16
            for d in range(D):
                vals = plsc.load_gather(
                    slab, [jnp.full((16,), d, jnp.int32), rel16])
                plsc.store_scatter(
                    stage, [pos16, jnp.full((16,), d, jnp.int32)], vals)
            return carry

        lax.fori_loop(0, nrows // 16, gather_16, 0)

        def sidx_one(k, carry):
            pos = k * 16 + iota16
            rows = hit_b[pl.ds(pl.multiple_of(k * 16, 16), 16)] * F + f
            plsc.store_scatter(sidx, [pos >> 7, pos & 127], rows)
            return carry

        lax.fori_loop(0, FLUSH // 16, sidx_one, 0)

        for k in range(FLUSH // 128):
            @pl.when(k * 128 < nrows)
            def _():
                pltpu.sync_copy(stage.at[pl.ds(k * 128, 128)],
                                out_ref.at[sidx.at[k]])

    for f in range(F):
        @pl.when(c == f // FPC)
        def _(f=f):
            wt = wt_refs[f]
            pltpu.sync_copy(wt.at[:, pl.ds(lo, SLABW)], slab)
            pltpu.sync_copy(feat_refs[f], idx_v)

            def scan_iter(i, nh):
                off = pl.multiple_of(i * 16, 16)
                v = idx_v[pl.ds(off, 16)]
                m = (v >= lo) & (v < hi)
                plsc.store_compressed(hit_i.at[pl.ds(nh, 16)], v - lo, mask=m)
                plsc.store_compressed(hit_b.at[pl.ds(nh, 16)],
                                      i * 16 + iota16, mask=m)
                nh = nh + plsc.all_reduce_population_count(m)[0]

                @pl.when(nh >= FLUSH)
                def _():
                    do_flush(f, FLUSH)
                    for t in range(2):
                        src = pl.ds(FLUSH + t * 16, 16)
                        dst = t * 16 + iota16
                        plsc.store_scatter(hit_i, [dst], hit_i[src])
                        plsc.store_scatter(hit_b, [dst], hit_b[src])

                return jnp.where(nh >= FLUSH, nh - FLUSH, nh)

            nh = lax.fori_loop(0, NITER, scan_iter, 0)

            @pl.when(nh > 0)
            def _():
                lane0 = iota16 == 0
                h0i = jnp.full(
                    (16,), jnp.sum(jnp.where(lane0, hit_i[pl.ds(0, 16)], 0)),
                    jnp.int32)
                h0b = jnp.full(
                    (16,), jnp.sum(jnp.where(lane0, hit_b[pl.ds(0, 16)], 0)),
                    jnp.int32)

                def pad_one(k, carry):
                    pos = k * 16 + iota16
                    pm = pos >= nh
                    plsc.store_scatter(hit_i, [pos], h0i, mask=pm)
                    plsc.store_scatter(hit_b, [pos], h0b, mask=pm)
                    return carry

                lax.fori_loop(0, FLUSH // 16, pad_one, 0)
                do_flush(f, ((nh + 127) >> 7) << 7)


@jax.jit
def _run(feats, wts):
    kern = functools.partial(
        pl.kernel,
        out_type=jax.ShapeDtypeStruct((F * B, D), jnp.float32),
        mesh=plsc.VectorSubcoreMesh(core_axis_name="c", subcore_axis_name="s"),
        scratch_types=[
            pltpu.VMEM((D, SLABW), jnp.float32),
            pltpu.VMEM((B,), jnp.int32),
            pltpu.VMEM((CAP,), jnp.int32),
            pltpu.VMEM((CAP,), jnp.int32),
            pltpu.VMEM((FLUSH, D), jnp.float32),
            pltpu.VMEM((FLUSH // 128, 128), jnp.int32),
        ],
        compiler_params=pltpu.CompilerParams(use_tc_tiling_on_sc=False, needs_layout_passes=False),
    )(_sc_body)
    out2 = kern(*feats, *wts)
    return out2.reshape(B, F * D)


def kernel(feat_0, feat_1, feat_2, feat_3, feat_4, feat_5, feat_6, feat_7,
           feat_8, feat_9, feat_10, feat_11, feat_12, feat_13, feat_14,
           feat_15, feat_16, feat_17, feat_18, feat_19, feat_20, feat_21,
           feat_22, feat_23, feat_24, feat_25,
           W_0, W_1, W_2, W_3, W_4, W_5, W_6, W_7, W_8, W_9, W_10, W_11,
           W_12, W_13, W_14, W_15, W_16, W_17, W_18, W_19, W_20, W_21,
           W_22, W_23, W_24, W_25):
    feats = [feat_0, feat_1, feat_2, feat_3, feat_4, feat_5, feat_6, feat_7,
             feat_8, feat_9, feat_10, feat_11, feat_12, feat_13, feat_14,
             feat_15, feat_16, feat_17, feat_18, feat_19, feat_20, feat_21,
             feat_22, feat_23, feat_24, feat_25]
    tables = [W_0, W_1, W_2, W_3, W_4, W_5, W_6, W_7, W_8, W_9, W_10, W_11,
              W_12, W_13, W_14, W_15, W_16, W_17, W_18, W_19, W_20, W_21,
              W_22, W_23, W_24, W_25]
    wts = [jnp.swapaxes(w, 0, 1) for w in tables]
    return _run(feats, wts)


# speedup vs baseline: 1.6979x; 1.2271x over previous
import functools

import jax
import jax.numpy as jnp
from jax import lax
from jax.experimental import pallas as pl
from jax.experimental.pallas import tpu as pltpu
from jax.experimental.pallas import tpu_sc as plsc

F = 26
V = 100000
D = 16
B = 16384
NC = 2
NS = 16
FPC = F // NC
CW = 6248
SLABW = CW + 32
CAP = 592
FLUSH = 512
NITER = B // 16


def _sc_body(*refs):
    feat_refs = refs[:F]
    wt_refs = refs[F:2 * F]
    out_ref = refs[2 * F]
    slab = refs[2 * F + 1]
    idx_v = refs[2 * F + 2]
    hit_i = refs[2 * F + 3]
    hit_b = refs[2 * F + 4]
    stage = refs[2 * F + 5]
    sidx = refs[2 * F + 6]

    c = lax.axis_index("c")
    s = lax.axis_index("s")
    lo = pl.multiple_of(s * CW, 8)
    hi = lo + CW + jnp.where(s == NS - 1, 32, 0)

    iota16 = lax.iota(jnp.int32, 16)

    def do_flush(f, nrows):
        def gather_16(k, carry):
            off = pl.multiple_of(k * 16, 16)
            rel16 = hit_i[pl.ds(off, 16)]
            pos16 = off + iota16
            for d in range(D):
                vals = plsc.load_gather(
                    slab, [jnp.full((16,), d, jnp.int32), rel16])
                plsc.store_scatter(
                    stage, [pos16, jnp.full((16,), d, jnp.int32)], vals)
            return carry

        lax.fori_loop(0, nrows // 16, gather_16, 0)

        def sidx_one(k, carry):
            pos = k * 16 + iota16
            rows = hit_b[pl.ds(pl.multiple_of(k * 16, 16), 16)] * F + f
            plsc.store_scatter(sidx, [pos >> 7, pos & 127], rows)
            return carry

        lax.fori_loop(0, FLUSH // 16, sidx_one, 0)

        for k in range(FLUSH // 128):
            @pl.when(k * 128 < nrows)
            def _():
                pltpu.sync_copy(stage.at[pl.ds(k * 128, 128)],
                                out_ref.at[sidx.at[k]])

    for f in range(F):
        @pl.when(c == f // FPC)
        def _(f=f):
            wt = wt_refs[f]
            pltpu.sync_copy(wt.at[:, pl.ds(lo, SLABW)], slab)
            pltpu.sync_copy(feat_refs[f], idx_v)

            def scan_iter(i, nh):
                off0 = pl.multiple_of(i * 64, 16)
                for u in range(4):
                    off = off0 + u * 16
                    v = idx_v[pl.ds(off, 16)]
                    m = (v >= lo) & (v < hi)
                    plsc.store_compressed(hit_i.at[pl.ds(nh, 16)],
                                          v - lo, mask=m)
                    plsc.store_compressed(hit_b.at[pl.ds(nh, 16)],
                                          off + iota16, mask=m)
                    nh = nh + plsc.all_reduce_population_count(m)[0]

                @pl.when(nh >= FLUSH)
                def _():
                    do_flush(f, FLUSH)
                    for t in range(2):
                        src = pl.ds(FLUSH + t * 16, 16)
                        dst = t * 16 + iota16
                        plsc.store_scatter(hit_i, [dst], hit_i[src])
                        plsc.store_scatter(hit_b, [dst], hit_b[src])

                return jnp.where(nh >= FLUSH, nh - FLUSH, nh)

            nh = lax.fori_loop(0, NITER // 4, scan_iter, 0)

            @pl.when(nh > 0)
            def _():
                lane0 = iota16 == 0
                h0i = jnp.full(
                    (16,), jnp.sum(jnp.where(lane0, hit_i[pl.ds(0, 16)], 0)),
                    jnp.int32)
                h0b = jnp.full(
                    (16,), jnp.sum(jnp.where(lane0, hit_b[pl.ds(0, 16)], 0)),
                    jnp.int32)

                def pad_one(k, carry):
                    pos = k * 16 + iota16
                    pm = pos >= nh
                    plsc.store_scatter(hit_i, [pos], h0i, mask=pm)
                    plsc.store_scatter(hit_b, [pos], h0b, mask=pm)
                    return carry

                lax.fori_loop(0, FLUSH // 16, pad_one, 0)
                do_flush(f, ((nh + 127) >> 7) << 7)


@jax.jit
def _run(feats, wts):
    kern = functools.partial(
        pl.kernel,
        out_type=jax.ShapeDtypeStruct((F * B, D), jnp.float32),
        mesh=plsc.VectorSubcoreMesh(core_axis_name="c", subcore_axis_name="s"),
        scratch_types=[
            pltpu.VMEM((D, SLABW), jnp.float32),
            pltpu.VMEM((B,), jnp.int32),
            pltpu.VMEM((CAP,), jnp.int32),
            pltpu.VMEM((CAP,), jnp.int32),
            pltpu.VMEM((FLUSH, D), jnp.float32),
            pltpu.VMEM((FLUSH // 128, 128), jnp.int32),
        ],
        compiler_params=pltpu.CompilerParams(use_tc_tiling_on_sc=False, needs_layout_passes=False),
    )(_sc_body)
    out2 = kern(*feats, *wts)
    return out2.reshape(B, F * D)


def kernel(feat_0, feat_1, feat_2, feat_3, feat_4, feat_5, feat_6, feat_7,
           feat_8, feat_9, feat_10, feat_11, feat_12, feat_13, feat_14,
           feat_15, feat_16, feat_17, feat_18, feat_19, feat_20, feat_21,
           feat_22, feat_23, feat_24, feat_25,
           W_0, W_1, W_2, W_3, W_4, W_5, W_6, W_7, W_8, W_9, W_10, W_11,
           W_12, W_13, W_14, W_15, W_16, W_17, W_18, W_19, W_20, W_21,
           W_22, W_23, W_24, W_25):
    feats = [feat_0, feat_1, feat_2, feat_3, feat_4, feat_5, feat_6, feat_7,
             feat_8, feat_9, feat_10, feat_11, feat_12, feat_13, feat_14,
             feat_15, feat_16, feat_17, feat_18, feat_19, feat_20, feat_21,
             feat_22, feat_23, feat_24, feat_25]
    tables = [W_0, W_1, W_2, W_3, W_4, W_5, W_6, W_7, W_8, W_9, W_10, W_11,
              W_12, W_13, W_14, W_15, W_16, W_17, W_18, W_19, W_20, W_21,
              W_22, W_23, W_24, W_25]
    wts = [jnp.swapaxes(w, 0, 1) for w in tables]
    return _run(feats, wts)


# speedup vs baseline: 1.7458x; 1.0282x over previous
import functools

import jax
import jax.numpy as jnp
from jax import lax
from jax.experimental import pallas as pl
from jax.experimental.pallas import tpu as pltpu
from jax.experimental.pallas import tpu_sc as plsc

F = 26
V = 100000
D = 16
B = 16384
NC = 2
NS = 16
FPC = F // NC
CW = 6248
SLABW = CW + 32
CAP = 656
FLUSH = 512
NITER = B // 16


def _sc_body(*refs):
    feat_refs = refs[:F]
    wt_refs = refs[F:2 * F]
    out_ref = refs[2 * F]
    slab = refs[2 * F + 1]
    idx_v = refs[2 * F + 2]
    hit_i = refs[2 * F + 3]
    hit_b = refs[2 * F + 4]
    stage = refs[2 * F + 5]
    sidx = refs[2 * F + 6]

    c = lax.axis_index("c")
    s = lax.axis_index("s")
    lo = pl.multiple_of(s * CW, 8)
    hi = lo + CW + jnp.where(s == NS - 1, 32, 0)

    iota16 = lax.iota(jnp.int32, 16)

    def do_flush(f, nrows):
        def gather_16(k, carry):
            off = pl.multiple_of(k * 16, 16)
            rel16 = hit_i[pl.ds(off, 16)]
            pos16 = off + iota16
            for d in range(D):
                vals = plsc.load_gather(
                    slab, [jnp.full((16,), d, jnp.int32), rel16])
                plsc.store_scatter(
                    stage, [pos16, jnp.full((16,), d, jnp.int32)], vals)
            return carry

        lax.fori_loop(0, nrows // 16, gather_16, 0)

        def sidx_one(k, carry):
            pos = k * 16 + iota16
            rows = hit_b[pl.ds(pl.multiple_of(k * 16, 16), 16)] * F + f
            plsc.store_scatter(sidx, [pos >> 7, pos & 127], rows)
            return carry

        lax.fori_loop(0, FLUSH // 16, sidx_one, 0)

        for k in range(FLUSH // 128):
            @pl.when(k * 128 < nrows)
            def _():
                pltpu.sync_copy(stage.at[pl.ds(k * 128, 128)],
                                out_ref.at[sidx.at[k]])

    for f in range(F):
        @pl.when(c == f // FPC)
        def _(f=f):
            wt = wt_refs[f]
            pltpu.sync_copy(wt.at[:, pl.ds(lo, SLABW)], slab)
            pltpu.sync_copy(feat_refs[f], idx_v)

            def scan_iter(i, nh):
                off0 = pl.multiple_of(i * 128, 16)
                for u in range(8):
                    off = off0 + u * 16
                    v = idx_v[pl.ds(off, 16)]
                    m = (v >= lo) & (v < hi)
                    plsc.store_compressed(hit_i.at[pl.ds(nh, 16)],
                                          v - lo, mask=m)
                    plsc.store_compressed(hit_b.at[pl.ds(nh, 16)],
                                          off + iota16, mask=m)
                    nh = nh + plsc.all_reduce_population_count(m)[0]

                @pl.when(nh >= FLUSH)
                def _():
                    do_flush(f, FLUSH)
                    for t in range(2):
                        src = pl.ds(FLUSH + t * 16, 16)
                        dst = t * 16 + iota16
                        plsc.store_scatter(hit_i, [dst], hit_i[src])
                        plsc.store_scatter(hit_b, [dst], hit_b[src])

                return jnp.where(nh >= FLUSH, nh - FLUSH, nh)

            nh = lax.fori_loop(0, NITER // 8, scan_iter, 0)

            @pl.when(nh > 0)
            def _():
                lane0 = iota16 == 0
                h0i = jnp.full(
                    (16,), jnp.sum(jnp.where(lane0, hit_i[pl.ds(0, 16)], 0)),
                    jnp.int32)
                h0b = jnp.full(
                    (16,), jnp.sum(jnp.where(lane0, hit_b[pl.ds(0, 16)], 0)),
                    jnp.int32)

                def pad_one(k, carry):
                    pos = k * 16 + iota16
                    pm = pos >= nh
                    plsc.store_scatter(hit_i, [pos], h0i, mask=pm)
                    plsc.store_scatter(hit_b, [pos], h0b, mask=pm)
                    return carry

                lax.fori_loop(0, FLUSH // 16, pad_one, 0)
                do_flush(f, ((nh + 127) >> 7) << 7)


@jax.jit
def _run(feats, wts):
    kern = functools.partial(
        pl.kernel,
        out_type=jax.ShapeDtypeStruct((F * B, D), jnp.float32),
        mesh=plsc.VectorSubcoreMesh(core_axis_name="c", subcore_axis_name="s"),
        scratch_types=[
            pltpu.VMEM((D, SLABW), jnp.float32),
            pltpu.VMEM((B,), jnp.int32),
            pltpu.VMEM((CAP,), jnp.int32),
            pltpu.VMEM((CAP,), jnp.int32),
            pltpu.VMEM((FLUSH, D), jnp.float32),
            pltpu.VMEM((FLUSH // 128, 128), jnp.int32),
        ],
        compiler_params=pltpu.CompilerParams(use_tc_tiling_on_sc=False, needs_layout_passes=False),
    )(_sc_body)
    out2 = kern(*feats, *wts)
    return out2.reshape(B, F * D)


def kernel(feat_0, feat_1, feat_2, feat_3, feat_4, feat_5, feat_6, feat_7,
           feat_8, feat_9, feat_10, feat_11, feat_12, feat_13, feat_14,
           feat_15, feat_16, feat_17, feat_18, feat_19, feat_20, feat_21,
           feat_22, feat_23, feat_24, feat_25,
           W_0, W_1, W_2, W_3, W_4, W_5, W_6, W_7, W_8, W_9, W_10, W_11,
           W_12, W_13, W_14, W_15, W_16, W_17, W_18, W_19, W_20, W_21,
           W_22, W_23, W_24, W_25):
    feats = [feat_0, feat_1, feat_2, feat_3, feat_4, feat_5, feat_6, feat_7,
             feat_8, feat_9, feat_10, feat_11, feat_12, feat_13, feat_14,
             feat_15, feat_16, feat_17, feat_18, feat_19, feat_20, feat_21,
             feat_22, feat_23, feat_24, feat_25]
    tables = [W_0, W_1, W_2, W_3, W_4, W_5, W_6, W_7, W_8, W_9, W_10, W_11,
              W_12, W_13, W_14, W_15, W_16, W_17, W_18, W_19, W_20, W_21,
              W_22, W_23, W_24, W_25]
    wts = [jnp.swapaxes(w, 0, 1) for w in tables]
    return _run(feats, wts)
